# Initial kernel scaffold; baseline (speedup 1.0000x reference)
import jax
import jax.numpy as jnp
from jax.experimental import pallas as pl


def kernel(x, edge_index, edge_attr, pe, rwse, batch, params):
    raise NotImplementedError("write your pallas kernel here")


# speedup vs baseline: 1.7321x; 1.7321x over previous
import functools

import numpy as np
import jax
import jax.numpy as jnp
from jax import lax
from jax.experimental import pallas as pl
from jax.experimental.pallas import tpu as pltpu
from jax.experimental.pallas import tpu_sc as plsc

N = 10000
E = 160000
D = 128
HEADS = 8
DH = 16
M = 64

W_SC = 32
CH = 128
NCH_W = 40
EP = W_SC * NCH_W * CH
NROW_CH = EP // CH
NP = 10240
ZR = NP // 16

_f32 = jnp.float32
_i32 = jnp.int32

@functools.lru_cache(maxsize=1)
def _sc_kernels():
    mesh = plsc.VectorSubcoreMesh(core_axis_name="c", subcore_axis_name="s")

    @functools.partial(
        pl.kernel,
        mesh=mesh,
        out_type=(
            jax.ShapeDtypeStruct((EP, D), _f32),
            jax.ShapeDtypeStruct((EP, D), _f32),
        ),
        scratch_types=[
            pltpu.VMEM((CH,), _i32),
            pltpu.VMEM((CH,), _i32),
            pltpu.VMEM((CH, D), _f32),
            pltpu.VMEM((CH, D), _f32),
            pltpu.SemaphoreType.DMA,
            pltpu.SemaphoreType.DMA,
        ],
    )
    def gather_k(a_hbm, b_hbm, dst_hbm, src_hbm, ga_hbm, gb_hbm,
                 idx_a, idx_b, rows_a, rows_b, sem_a, sem_b):
        wid = lax.axis_index("s") * 2 + lax.axis_index("c")
        base0 = wid * (NCH_W * CH)

        @pl.loop(0, NCH_W)
        def _(c):
            base = base0 + c * CH
            pltpu.sync_copy(dst_hbm.at[pl.ds(base, CH)], idx_a)
            pltpu.sync_copy(src_hbm.at[pl.ds(base, CH)], idx_b)
            ca = pltpu.async_copy(a_hbm.at[idx_a], rows_a, sem_a)
            cb = pltpu.async_copy(b_hbm.at[idx_b], rows_b, sem_b)
            ca.wait()
            cb.wait()
            pltpu.sync_copy(rows_a, ga_hbm.at[pl.ds(base, CH)])
            pltpu.sync_copy(rows_b, gb_hbm.at[pl.ds(base, CH)])

    @functools.partial(
        pl.kernel,
        mesh=mesh,
        out_type=jax.ShapeDtypeStruct((2 * NP, D), _f32),
        scratch_types=[
            pltpu.VMEM((1, CH), _i32),
            pltpu.VMEM((CH, D), _f32),
            pltpu.VMEM_SHARED((NP, D), _f32),
            pltpu.SemaphoreType.DMA,
        ],
    )
    def scatter_k(m_hbm, sidx_hbm, zeros_hbm, out_hbm,
                  idx_v, rows_v, shared, sem):
        core = lax.axis_index("c")
        sub = lax.axis_index("s")
        wid = sub * 2 + core
        pltpu.sync_copy(zeros_hbm, shared.at[pl.ds(sub * ZR, ZR)])
        plsc.subcore_barrier()

        @pl.loop(0, NCH_W)
        def _(c):
            ch = wid * NCH_W + c
            pltpu.sync_copy(m_hbm.at[pl.ds(ch * CH, CH)], rows_v)
            pltpu.sync_copy(sidx_hbm.at[ch], idx_v)
            pltpu.sync_copy(rows_v, shared.at[idx_v.at[0]], add=True)

        plsc.subcore_barrier()
        pltpu.sync_copy(shared.at[pl.ds(sub * ZR, ZR)],
                        out_hbm.at[pl.ds(core * NP + sub * ZR, ZR)])

    return gather_k, scatter_k


def _sc_gather(a, b, gdst, gsrc):
    return _sc_kernels()[0](a, b, gdst, gsrc)


def _sc_scatter(m, sidx, zeros):
    return _sc_kernels()[1](m, sidx, zeros)


_NBE = 20
_BE_STATS = EP // _NBE


def _stats_body(ea_ref, out_ref):
    i = pl.program_id(0)
    blk = ea_ref[...]
    s = jnp.sum(blk)
    s2 = jnp.sum(blk * blk)

    @pl.when(i == 0)
    def _():
        out_ref[0, 0] = s
        out_ref[0, 1] = s2

    @pl.when(i > 0)
    def _():
        out_ref[0, 0] += s
        out_ref[0, 1] += s2


def _stats_call(ea_pad):
    return pl.pallas_call(
        _stats_body,
        grid=(_NBE,),
        in_specs=[pl.BlockSpec((_BE_STATS, 1), lambda i: (i, 0))],
        out_specs=pl.BlockSpec((1, 2), lambda i: (0, 0),
                               memory_space=pltpu.SMEM),
        out_shape=jax.ShapeDtypeStruct((1, 2), _f32),
    )(ea_pad)


def _ea_body(ea_ref, st_ref, w_ref, g_ref, b_ref, out_ref):
    mu = st_ref[0, 0] / E
    va = st_ref[0, 1] / E - mu * mu
    w = w_ref[...]
    alpha = w * g_ref[...] / jnp.sqrt(w * w * va + 1e-5)
    out_ref[...] = jnp.maximum((ea_ref[...] - mu) * alpha + b_ref[...], 0.0)


def _ea_call(ea_pad, stats, w, g, b):
    return pl.pallas_call(
        _ea_body,
        grid=(_NBE,),
        in_specs=[
            pl.BlockSpec((_BE_STATS, 1), lambda i: (i, 0)),
            pl.BlockSpec((1, 2), lambda i: (0, 0), memory_space=pltpu.SMEM),
            pl.BlockSpec((1, D), lambda i: (0, 0)),
            pl.BlockSpec((1, D), lambda i: (0, 0)),
            pl.BlockSpec((1, D), lambda i: (0, 0)),
        ],
        out_specs=pl.BlockSpec((_BE_STATS, D), lambda i: (i, 0)),
        out_shape=jax.ShapeDtypeStruct((EP, D), _f32),
    )(ea_pad, stats, w, g, b)


def _enc_body(x_ref, rwse_ref, wn_ref, g1_ref, b1_ref,
              wr_ref, br_ref, g2_ref, b2_ref,
              wp_ref, bp_ref, gp_ref, bpp_ref,
              w1i_ref, w1j_ref,
              h_ref, a_ref, b_ref):
    xv = x_ref[...]
    mu = jnp.mean(xv)
    va = jnp.mean((xv - mu) ** 2)
    wn = wn_ref[...]
    alpha = wn * g1_ref[...] / jnp.sqrt(wn * wn * va + 1e-5)
    h1 = jnp.maximum((xv - mu) * alpha + b1_ref[...], 0.0)

    y2 = jnp.dot(rwse_ref[...], wr_ref[...],
                 preferred_element_type=_f32) + br_ref[...]
    m2 = jnp.mean(y2, axis=0, keepdims=True)
    v2 = jnp.mean((y2 - m2) ** 2, axis=0, keepdims=True)
    h2 = jnp.maximum((y2 - m2) / jnp.sqrt(v2 + 1e-5) * g2_ref[...]
                     + b2_ref[...], 0.0)
    h = h1 + h2

    z = jnp.maximum(jnp.dot(h, wp_ref[...], preferred_element_type=_f32)
                    + bp_ref[...], 0.0)
    m3 = jnp.mean(z, axis=0, keepdims=True)
    v3 = jnp.mean((z - m3) ** 2, axis=0, keepdims=True)
    h0 = (z - m3) / jnp.sqrt(v3 + 1e-5) * gp_ref[...] + bpp_ref[...]

    h_ref[...] = h0
    a_ref[...] = jnp.dot(h0, w1i_ref[...], preferred_element_type=_f32)
    b_ref[...] = jnp.dot(h0, w1j_ref[...], preferred_element_type=_f32)


def _enc_call(x, rwse, wn, g1, b1, wr, br, g2, b2, wp, bp, gp, bpp, w1i, w1j):
    return pl.pallas_call(
        _enc_body,
        out_shape=(
            jax.ShapeDtypeStruct((N, D), _f32),
            jax.ShapeDtypeStruct((N, D), _f32),
            jax.ShapeDtypeStruct((N, D), _f32),
        ),
    )(x, rwse, wn, g1, b1, wr, br, g2, b2, wp, bp, gp, bpp, w1i, w1j)


_BEM = 2048
_NBEM = EP // _BEM


def _edge_body(ga_ref, gb_ref, ea_ref, w1e_ref, b1_ref, w2_ref, b2_ref,
               m_ref):
    pre = (ga_ref[...] + gb_ref[...]
           + jnp.dot(ea_ref[...], w1e_ref[...], preferred_element_type=_f32)
           + b1_ref[...])
    m_ref[...] = jnp.dot(jnp.maximum(pre, 0.0), w2_ref[...],
                         preferred_element_type=_f32) + b2_ref[...]


def _edge_call(ga, gb, ea, w1e, b1, w2, b2):
    blk = lambda i: (i, 0)
    cst = lambda i: (0, 0)
    return pl.pallas_call(
        _edge_body,
        grid=(_NBEM,),
        in_specs=[
            pl.BlockSpec((_BEM, D), blk),
            pl.BlockSpec((_BEM, D), blk),
            pl.BlockSpec((_BEM, D), blk),
            pl.BlockSpec((D, D), cst),
            pl.BlockSpec((1, D), cst),
            pl.BlockSpec((D, D), cst),
            pl.BlockSpec((1, D), cst),
        ],
        out_specs=pl.BlockSpec((_BEM, D), blk),
        out_shape=jax.ShapeDtypeStruct((EP, D), _f32),
    )(ga, gb, ea, w1e, b1, w2, b2)


_CHR = 1000
_NBR = N // _CHR

_DN = DH ** -0.25
_RATIO = M ** -0.5

_EYE8 = np.eye(8, dtype=np.float32)
_R8 = np.kron(_EYE8, np.ones((1, 64), np.float32))
_HD = np.kron(_EYE8, np.ones((16, 1), np.float32))
_S8 = np.kron(_EYE8, np.ones((1, 16), np.float32))
_R8T = np.kron(_EYE8, np.ones((64, 1), np.float32))
_CTXM = (_EYE8[:, None, :, None]
         * np.ones((1, 64, 1, 16), np.float32)).reshape(512, 128)


def _ln(x, g, b):
    mu = jnp.mean(x, axis=-1, keepdims=True)
    va = jnp.mean((x - mu) ** 2, axis=-1, keepdims=True)
    return (x - mu) / jnp.sqrt(va + 1e-5) * g + b


def _mm(a, b):
    return jnp.dot(a, b, preferred_element_type=_f32)


def _phi_raw(hc, t_w, bias, p_mat, hd_mat, r8_mat):
    t = (_mm(hc, t_w) + bias) * _DN
    u = _mm(t, p_mat)
    d2 = _mm(t * t, hd_mat) * 0.5
    return u, _mm(d2, r8_mat)


_CBLK = lambda i: (i, 0)
_CCST = lambda i: (0, 0)


def _kmax_body(h_ref, kt, bk, p_mat, out_ref):
    i = pl.program_id(0)
    t = (_mm(h_ref[...], kt[...]) + bk[...]) * _DN
    m = jnp.max(_mm(t, p_mat[...]))

    @pl.when(i == 0)
    def _():
        out_ref[0, 0] = m

    @pl.when(i > 0)
    def _():
        out_ref[0, 0] = jnp.maximum(out_ref[0, 0], m)


def _kmax_call(h, kt, bk, p_mat):
    return pl.pallas_call(
        _kmax_body,
        grid=(_NBR,),
        in_specs=[
            pl.BlockSpec((_CHR, D), _CBLK),
            pl.BlockSpec((D, D), _CCST),
            pl.BlockSpec((1, D), _CCST),
            pl.BlockSpec((D, 512), _CCST),
        ],
        out_specs=pl.BlockSpec((1, 1), _CCST, memory_space=pltpu.SMEM),
        out_shape=jax.ShapeDtypeStruct((1, 1), _f32),
    )(h, kt, bk, p_mat)


def _ctx_body(h_ref, kt, bk, vt, bv, p_mat, hd_mat, r8_mat, mx_ref,
              ctx_ref, ksum_ref):
    i = pl.program_id(0)
    hc = h_ref[...]
    u, d2b = _phi_raw(hc, kt[...], bk[...], p_mat[...], hd_mat[...],
                      r8_mat[...])
    kp = _RATIO * (jnp.exp(u - d2b - mx_ref[0, 0]) + 1e-4)
    vc = _mm(hc, vt[...]) + bv[...]
    c = lax.dot_general(kp, vc, (((0,), (0,)), ((), ())),
                        preferred_element_type=_f32)
    ks = jnp.sum(kp, axis=0, keepdims=True)

    @pl.when(i == 0)
    def _():
        ctx_ref[...] = c
        ksum_ref[...] = ks

    @pl.when(i > 0)
    def _():
        ctx_ref[...] += c
        ksum_ref[...] += ks


def _ctx_call(h, kt, bk, vt, bv, p_mat, mx):
    return pl.pallas_call(
        _ctx_body,
        grid=(_NBR,),
        in_specs=[
            pl.BlockSpec((_CHR, D), _CBLK),
            pl.BlockSpec((D, D), _CCST),
            pl.BlockSpec((1, D), _CCST),
            pl.BlockSpec((D, D), _CCST),
            pl.BlockSpec((1, D), _CCST),
            pl.BlockSpec((D, 512), _CCST),
            pl.BlockSpec((D, 8), _CCST),
            pl.BlockSpec((8, 512), _CCST),
            pl.BlockSpec((1, 1), _CCST, memory_space=pltpu.SMEM),
        ],
        out_specs=(
            pl.BlockSpec((512, D), _CCST),
            pl.BlockSpec((1, 512), _CCST),
        ),
        out_shape=(
            jax.ShapeDtypeStruct((512, D), _f32),
            jax.ShapeDtypeStruct((1, 512), _f32),
        ),
    )(h, kt, bk, vt, bv, p_mat, jnp.asarray(_HD), jnp.asarray(_R8), mx)


def _main_core(hc, p0c, p1c, ctx, ksum, w):
    (n1a, n1b, bn1, n2t, bn2, ggcn, bgcn, glo, blo,
     qt, bq, ot, bo, gat, bat,
     f1t, bf1, f2t, bf2, gff, bff,
     p_mat, hd_mat, r8_mat, r8t_mat, s8_mat, ctxm_mat) = w
    u, d2b = _phi_raw(hc, qt, bq, p_mat, hd_mat, r8_mat)
    mxq = jnp.concatenate(
        [jnp.max(u[:, hh * 64:(hh + 1) * 64], axis=1, keepdims=True)
         for hh in range(HEADS)], axis=1)
    qp = _RATIO * (jnp.exp(u - d2b - _mm(mxq, r8_mat)) + 1e-4)
    num = _mm(qp, ctx * ctxm_mat)
    den = _mm(qp * ksum, r8t_mat) + 1e-6
    o = num / _mm(den, s8_mat)
    attn = _mm(o, ot) + bo
    h_attn = _ln(hc + attn, gat, bat)

    agg = p0c + p1c
    t = jnp.maximum(_mm(hc, n1a) + _mm(agg, n1b) + bn1, 0.0)
    out = _mm(t, n2t) + bn2 + hc
    h_local = _ln(out, ggcn, bgcn)
    h_local = _ln(hc + h_local, glo, blo)

    hs = h_local + h_attn
    ff = _mm(jnp.maximum(_mm(hs, f1t) + bf1, 0.0), f2t) + bf2
    return _ln(hs + ff, gff, bff)


_NW_DENSE = 27


def _main_mid_body(*refs):
    (h_ref, p0_ref, p1_ref, ctx_ref, ksum_ref) = refs[:5]
    w = tuple(r[...] for r in refs[5:5 + _NW_DENSE])
    w1in, w1jn = refs[5 + _NW_DENSE], refs[6 + _NW_DENSE]
    h_out, a_out, b_out = refs[-3:]
    hn = _main_core(h_ref[...], p0_ref[...], p1_ref[...], ctx_ref[...],
                    ksum_ref[...], w)
    h_out[...] = hn
    a_out[...] = _mm(hn, w1in[...])
    b_out[...] = _mm(hn, w1jn[...])


def _main_last_body(*refs):
    (h_ref, p0_ref, p1_ref, ctx_ref, ksum_ref) = refs[:5]
    w = tuple(r[...] for r in refs[5:5 + _NW_DENSE])
    h1t, bh1, h2t, bh2 = refs[5 + _NW_DENSE:9 + _NW_DENSE]
    y_out = refs[-1]
    hn = _main_core(h_ref[...], p0_ref[...], p1_ref[...], ctx_ref[...],
                    ksum_ref[...], w)
    z = jnp.maximum(_mm(hn, h1t[...]) + bh1[...], 0.0)
    y_out[...] = _mm(z, h2t[...]) + bh2[...]


def _dense_w_specs():
    return [
        pl.BlockSpec((D, D), _CCST),
        pl.BlockSpec((D, D), _CCST),
        pl.BlockSpec((1, D), _CCST),
        pl.BlockSpec((D, D), _CCST),
        pl.BlockSpec((1, D), _CCST),
        pl.BlockSpec((1, D), _CCST),
        pl.BlockSpec((1, D), _CCST),
        pl.BlockSpec((1, D), _CCST),
        pl.BlockSpec((1, D), _CCST),
        pl.BlockSpec((D, D), _CCST),
        pl.BlockSpec((1, D), _CCST),
        pl.BlockSpec((D, D), _CCST),
        pl.BlockSpec((1, D), _CCST),
        pl.BlockSpec((1, D), _CCST),
        pl.BlockSpec((1, D), _CCST),
        pl.BlockSpec((D, 256), _CCST),
        pl.BlockSpec((1, 256), _CCST),
        pl.BlockSpec((256, D), _CCST),
        pl.BlockSpec((1, D), _CCST),
        pl.BlockSpec((1, D), _CCST),
        pl.BlockSpec((1, D), _CCST),
        pl.BlockSpec((D, 512), _CCST),
        pl.BlockSpec((D, 8), _CCST),
        pl.BlockSpec((8, 512), _CCST),
        pl.BlockSpec((512, 8), _CCST),
        pl.BlockSpec((8, D), _CCST),
        pl.BlockSpec((512, D), _CCST),
    ]


def _dense_mid(h, p0, p1, ctx, ksum, wl):
    in_specs = [
        pl.BlockSpec((_CHR, D), _CBLK),
        pl.BlockSpec((_CHR, D), _CBLK),
        pl.BlockSpec((_CHR, D), _CBLK),
        pl.BlockSpec((512, D), _CCST),
        pl.BlockSpec((1, 512), _CCST),
    ] + _dense_w_specs() + [
        pl.BlockSpec((D, D), _CCST),
        pl.BlockSpec((D, D), _CCST),
    ]
    return pl.pallas_call(
        _main_mid_body,
        grid=(_NBR,),
        in_specs=in_specs,
        out_specs=(
            pl.BlockSpec((_CHR, D), _CBLK),
            pl.BlockSpec((_CHR, D), _CBLK),
            pl.BlockSpec((_CHR, D), _CBLK),
        ),
        out_shape=(
            jax.ShapeDtypeStruct((N, D), _f32),
            jax.ShapeDtypeStruct((N, D), _f32),
            jax.ShapeDtypeStruct((N, D), _f32),
        ),
    )(h, p0, p1, ctx, ksum, *wl)


def _dense_last(h, p0, p1, ctx, ksum, wl):
    in_specs = [
        pl.BlockSpec((_CHR, D), _CBLK),
        pl.BlockSpec((_CHR, D), _CBLK),
        pl.BlockSpec((_CHR, D), _CBLK),
        pl.BlockSpec((512, D), _CCST),
        pl.BlockSpec((1, 512), _CCST),
    ] + _dense_w_specs() + [
        pl.BlockSpec((D, D), _CCST),
        pl.BlockSpec((1, D), _CCST),
        pl.BlockSpec((D, 1), _CCST),
        pl.BlockSpec((1, 1), _CCST),
    ]
    return pl.pallas_call(
        _main_last_body,
        grid=(_NBR,),
        in_specs=in_specs,
        out_specs=pl.BlockSpec((_CHR, 1), _CBLK),
        out_shape=jax.ShapeDtypeStruct((N, 1), _f32),
    )(h, p0, p1, ctx, ksum, *wl)


def _row(v):
    return v.reshape(1, -1)


def kernel(x, edge_index, edge_attr, pe, rwse, batch, params):
    p = params
    src = edge_index[0].astype(_i32)
    dst = edge_index[1].astype(_i32)
    pad = EP - E
    zpad = jnp.zeros((pad,), _i32)
    gdst = jnp.concatenate([dst, zpad])
    gsrc = jnp.concatenate([src, zpad])
    sdst = jnp.concatenate([dst, jnp.full((pad,), N, _i32)]).reshape(
        NROW_CH, 1, CH)
    ea_pad = jnp.concatenate(
        [edge_attr, jnp.zeros((pad, 1), _f32)], axis=0)
    zeros_z = jnp.zeros((ZR, D), _f32)

    proj = p["proj"]
    p_mat = (_EYE8[:, None, :, None]
             * proj.T[None, :, None, :]).reshape(D, 512)

    stats = _stats_call(ea_pad)
    ea = _ea_call(ea_pad, stats,
                  _row(p["enc_edge"]["W"][:, 0]),
                  _row(p["enc_edge_bn"]["g"]), _row(p["enc_edge_bn"]["b"]))

    l0 = p["layers"][0]
    h, a_t, b_t = _enc_call(
        x, rwse,
        _row(p["enc_node"]["W"][:, 0]),
        _row(p["enc_node_bn"]["g"]), _row(p["enc_node_bn"]["b"]),
        p["enc_rwse"]["W"].T, _row(p["enc_rwse"]["b"]),
        _row(p["enc_rwse_bn"]["g"]), _row(p["enc_rwse_bn"]["b"]),
        p["pre"]["W"].T, _row(p["pre"]["b"]),
        _row(p["pre_bn"]["g"]), _row(p["pre_bn"]["b"]),
        l0["e1"]["W"][:, :D].T, l0["e1"]["W"][:, D:2 * D].T,
    )

    out = None
    for li in range(6):
        lp = p["layers"][li]
        ga, gb = _sc_gather(a_t, b_t, gdst, gsrc)
        m = _edge_call(ga, gb, ea,
                       lp["e1"]["W"][:, 2 * D:].T, _row(lp["e1"]["b"]),
                       lp["e2"]["W"].T, _row(lp["e2"]["b"]))
        parts = _sc_scatter(m, sdst, zeros_z)
        p0 = parts[:NP]
        p1 = parts[NP:]

        kt = lp["k"]["W"].T
        bk = _row(lp["k"]["b"])
        mx = _kmax_call(h, kt, bk, p_mat)
        ctx, ksum = _ctx_call(h, kt, bk, lp["v"]["W"].T, _row(lp["v"]["b"]),
                              p_mat, mx)

        wl = [
            lp["n1"]["W"][:, :D].T, lp["n1"]["W"][:, D:].T,
            _row(lp["n1"]["b"]),
            lp["n2"]["W"].T, _row(lp["n2"]["b"]),
            _row(lp["gcn_ln"]["g"]), _row(lp["gcn_ln"]["b"]),
            _row(lp["ln_local"]["g"]), _row(lp["ln_local"]["b"]),
            lp["q"]["W"].T, _row(lp["q"]["b"]),
            lp["o"]["W"].T, _row(lp["o"]["b"]),
            _row(lp["ln_attn"]["g"]), _row(lp["ln_attn"]["b"]),
            lp["f1"]["W"].T, _row(lp["f1"]["b"]),
            lp["f2"]["W"].T, _row(lp["f2"]["b"]),
            _row(lp["ln_ffn"]["g"]), _row(lp["ln_ffn"]["b"]),
            p_mat, _HD, _R8, _R8T, _S8, _CTXM,
        ]
        if li < 5:
            ln = p["layers"][li + 1]
            wl += [ln["e1"]["W"][:, :D].T, ln["e1"]["W"][:, D:2 * D].T]
            h, a_t, b_t = _dense_mid(h, p0, p1, ctx, ksum, wl)
        else:
            wl += [p["head1"]["W"].T, _row(p["head1"]["b"]),
                   p["head2"]["W"].T.reshape(D, 1),
                   _row(p["head2"]["b"]).reshape(1, 1)]
            out = _dense_last(h, p0, p1, ctx, ksum, wl)
    return out


# speedup vs baseline: 2.2555x; 1.3022x over previous
import functools

import numpy as np
import jax
import jax.numpy as jnp
from jax import lax
from jax.experimental import pallas as pl
from jax.experimental.pallas import tpu as pltpu
from jax.experimental.pallas import tpu_sc as plsc

N = 10000
E = 160000
D = 128
HEADS = 8
DH = 16
M = 64

W_SC = 32
CHG = 64
NCHG = 80
CHS = 128
NCHS = 40
EP = W_SC * NCHG * CHG
NROW_CH = EP // CHS
NP = 10240
ZR = NP // 16

_f32 = jnp.float32
_i32 = jnp.int32

@functools.lru_cache(maxsize=1)
def _sc_kernels():
    mesh = plsc.VectorSubcoreMesh(core_axis_name="c", subcore_axis_name="s")

    @functools.partial(
        pl.kernel,
        mesh=mesh,
        out_type=jax.ShapeDtypeStruct((EP, D), _f32),
        scratch_types=[
            pltpu.VMEM((NCHG * CHG,), _i32),
            pltpu.VMEM((NCHG * CHG,), _i32),
        ] + [pltpu.VMEM((CHG, D), _f32)] * 8
          + [pltpu.SemaphoreType.DMA] * 8,
    )
    def gather_k(a_hbm, b_hbm, dst_hbm, src_hbm, gs_hbm, *scr):
        idxd, idxs = scr[0], scr[1]
        bufa = list(scr[2:6])
        bufb = list(scr[6:10])
        sg = list(scr[10:14])
        sw = list(scr[14:18])
        wid = lax.axis_index("s") * 2 + lax.axis_index("c")
        base0 = wid * (NCHG * CHG)
        pltpu.sync_copy(dst_hbm.at[pl.ds(base0, NCHG * CHG)], idxd)
        pltpu.sync_copy(src_hbm.at[pl.ds(base0, NCHG * CHG)], idxs)

        def fire_gather(c, b):
            pltpu.async_copy(a_hbm.at[idxd.at[pl.ds(c * CHG, CHG)]],
                             bufa[b], sg[b])
            pltpu.async_copy(b_hbm.at[idxs.at[pl.ds(c * CHG, CHG)]],
                             bufb[b], sg[b])

        def wait_gather(b):
            pltpu.make_async_copy(a_hbm.at[pl.ds(0, CHG)], bufa[b],
                                  sg[b]).wait()
            pltpu.make_async_copy(a_hbm.at[pl.ds(0, CHG)], bufb[b],
                                  sg[b]).wait()

        def add_rows(b):
            ba, bb = bufa[b], bufb[b]

            @pl.loop(0, CHG)
            def _(r):
                for j in range(8):
                    sl = pl.ds(j * 16, 16)
                    ba[r, sl] += bb[r, sl]

        def fire_write(c, b):
            pltpu.async_copy(bufa[b], gs_hbm.at[pl.ds(base0 + c * CHG, CHG)],
                             sw[b])

        def wait_write(b):
            pltpu.make_async_copy(bufa[b], gs_hbm.at[pl.ds(0, CHG)],
                                  sw[b]).wait()

        for c in range(3):
            fire_gather(c, c)
        fire_gather(3, 3)
        wait_gather(0)
        add_rows(0)
        fire_write(0, 0)

        @pl.loop(0, 19)
        def _(t):
            for j in range(4):
                b = (1 + j) % 4
                b3 = j % 4
                c = 4 * t + 1 + j
                wait_write(b3)
                fire_gather_c = c + 3
                pltpu.async_copy(
                    a_hbm.at[idxd.at[pl.ds(fire_gather_c * CHG, CHG)]],
                    bufa[b3], sg[b3])
                pltpu.async_copy(
                    b_hbm.at[idxs.at[pl.ds(fire_gather_c * CHG, CHG)]],
                    bufb[b3], sg[b3])
                wait_gather(b)
                add_rows(b)
                pltpu.async_copy(bufa[b],
                                 gs_hbm.at[pl.ds(base0 + c * CHG, CHG)],
                                 sw[b])

        for c in (77, 78, 79):
            b = c % 4
            wait_gather(b)
            add_rows(b)
            fire_write(c, b)
        for b in range(4):
            wait_write(b)

    @functools.partial(
        pl.kernel,
        mesh=mesh,
        out_type=jax.ShapeDtypeStruct((2 * NP, D), _f32),
        scratch_types=[
            pltpu.VMEM((NCHS, 1, CHS), _i32),
            pltpu.VMEM((CHS, D), _f32),
            pltpu.VMEM((CHS, D), _f32),
            pltpu.VMEM_SHARED((NP, D), _f32),
            pltpu.SemaphoreType.DMA,
            pltpu.SemaphoreType.DMA,
        ],
    )
    def scatter_k(m_hbm, sidx_hbm, zeros_hbm, out_hbm,
                  idx_all, m0, m1, shared, s0, s1):
        core = lax.axis_index("c")
        sub = lax.axis_index("s")
        wid = sub * 2 + core
        bufs = (m0, m1)
        sems = (s0, s1)
        pltpu.sync_copy(zeros_hbm, shared.at[pl.ds(sub * ZR, ZR)])
        pltpu.sync_copy(sidx_hbm.at[pl.ds(wid * NCHS, NCHS)], idx_all)
        plsc.subcore_barrier()

        base0 = wid * NCHS * CHS

        def fire_load(c, b):
            pltpu.async_copy(m_hbm.at[pl.ds(base0 + c * CHS, CHS)],
                             bufs[b], sems[b])

        def wait_load(b):
            pltpu.make_async_copy(m_hbm.at[pl.ds(0, CHS)], bufs[b],
                                  sems[b]).wait()

        fire_load(0, 0)
        fire_load(1, 1)

        @pl.loop(0, 19)
        def _(t):
            for b in range(2):
                c = 2 * t + b
                wait_load(b)
                pltpu.sync_copy(bufs[b], shared.at[idx_all.at[c].at[0]],
                                add=True)
                pltpu.async_copy(m_hbm.at[pl.ds(base0 + (c + 2) * CHS, CHS)],
                                 bufs[b], sems[b])

        for c in (38, 39):
            b = c % 2
            wait_load(b)
            pltpu.sync_copy(bufs[b], shared.at[idx_all.at[c].at[0]],
                            add=True)

        plsc.subcore_barrier()
        pltpu.sync_copy(shared.at[pl.ds(sub * ZR, ZR)],
                        out_hbm.at[pl.ds(core * NP + sub * ZR, ZR)])

    return gather_k, scatter_k


def _sc_gather(a, b, gdst, gsrc):
    return _sc_kernels()[0](a, b, gdst, gsrc)


def _sc_scatter(m, sidx, zeros):
    return _sc_kernels()[1](m, sidx, zeros)


_NBE = 20
_BE_STATS = EP // _NBE


def _stats_body(ea_ref, out_ref):
    i = pl.program_id(0)
    blk = ea_ref[...]
    s = jnp.sum(blk)
    s2 = jnp.sum(blk * blk)

    @pl.when(i == 0)
    def _():
        out_ref[0, 0] = s
        out_ref[0, 1] = s2

    @pl.when(i > 0)
    def _():
        out_ref[0, 0] += s
        out_ref[0, 1] += s2


def _stats_call(ea_pad):
    return pl.pallas_call(
        _stats_body,
        grid=(_NBE,),
        in_specs=[pl.BlockSpec((_BE_STATS, 1), lambda i: (i, 0))],
        out_specs=pl.BlockSpec((1, 2), lambda i: (0, 0),
                               memory_space=pltpu.SMEM),
        out_shape=jax.ShapeDtypeStruct((1, 2), _f32),
    )(ea_pad)


def _ea_body(ea_ref, st_ref, w_ref, g_ref, b_ref, out_ref):
    mu = st_ref[0, 0] / E
    va = st_ref[0, 1] / E - mu * mu
    w = w_ref[...]
    alpha = w * g_ref[...] / jnp.sqrt(w * w * va + 1e-5)
    out_ref[...] = jnp.maximum((ea_ref[...] - mu) * alpha + b_ref[...], 0.0)


def _ea_call(ea_pad, stats, w, g, b):
    return pl.pallas_call(
        _ea_body,
        grid=(_NBE,),
        in_specs=[
            pl.BlockSpec((_BE_STATS, 1), lambda i: (i, 0)),
            pl.BlockSpec((1, 2), lambda i: (0, 0), memory_space=pltpu.SMEM),
            pl.BlockSpec((1, D), lambda i: (0, 0)),
            pl.BlockSpec((1, D), lambda i: (0, 0)),
            pl.BlockSpec((1, D), lambda i: (0, 0)),
        ],
        out_specs=pl.BlockSpec((_BE_STATS, D), lambda i: (i, 0)),
        out_shape=jax.ShapeDtypeStruct((EP, D), _f32),
    )(ea_pad, stats, w, g, b)


def _enc_body(x_ref, rwse_ref, wn_ref, g1_ref, b1_ref,
              wr_ref, br_ref, g2_ref, b2_ref,
              wp_ref, bp_ref, gp_ref, bpp_ref,
              w1i_ref, w1j_ref,
              h_ref, a_ref, b_ref):
    xv = x_ref[...]
    mu = jnp.mean(xv)
    va = jnp.mean((xv - mu) ** 2)
    wn = wn_ref[...]
    alpha = wn * g1_ref[...] / jnp.sqrt(wn * wn * va + 1e-5)
    h1 = jnp.maximum((xv - mu) * alpha + b1_ref[...], 0.0)

    y2 = jnp.dot(rwse_ref[...], wr_ref[...],
                 preferred_element_type=_f32) + br_ref[...]
    m2 = jnp.mean(y2, axis=0, keepdims=True)
    v2 = jnp.mean((y2 - m2) ** 2, axis=0, keepdims=True)
    h2 = jnp.maximum((y2 - m2) / jnp.sqrt(v2 + 1e-5) * g2_ref[...]
                     + b2_ref[...], 0.0)
    h = h1 + h2

    z = jnp.maximum(jnp.dot(h, wp_ref[...], preferred_element_type=_f32)
                    + bp_ref[...], 0.0)
    m3 = jnp.mean(z, axis=0, keepdims=True)
    v3 = jnp.mean((z - m3) ** 2, axis=0, keepdims=True)
    h0 = (z - m3) / jnp.sqrt(v3 + 1e-5) * gp_ref[...] + bpp_ref[...]

    h_ref[...] = h0
    a_ref[...] = jnp.dot(h0, w1i_ref[...], preferred_element_type=_f32)
    b_ref[...] = jnp.dot(h0, w1j_ref[...], preferred_element_type=_f32)


def _enc_call(x, rwse, wn, g1, b1, wr, br, g2, b2, wp, bp, gp, bpp, w1i, w1j):
    return pl.pallas_call(
        _enc_body,
        out_shape=(
            jax.ShapeDtypeStruct((N, D), _f32),
            jax.ShapeDtypeStruct((N, D), _f32),
            jax.ShapeDtypeStruct((N, D), _f32),
        ),
    )(x, rwse, wn, g1, b1, wr, br, g2, b2, wp, bp, gp, bpp, w1i, w1j)


_BEM = 2048
_NBEM = EP // _BEM


def _edge_body(gs_ref, ea_ref, w1e_ref, b1_ref, w2_ref, b2_ref, m_ref):
    pre = (gs_ref[...]
           + jnp.dot(ea_ref[...], w1e_ref[...], preferred_element_type=_f32)
           + b1_ref[...])
    m_ref[...] = jnp.dot(jnp.maximum(pre, 0.0), w2_ref[...],
                         preferred_element_type=_f32) + b2_ref[...]


def _edge_call(gs, ea, w1e, b1, w2, b2):
    blk = lambda i: (i, 0)
    cst = lambda i: (0, 0)
    return pl.pallas_call(
        _edge_body,
        grid=(_NBEM,),
        in_specs=[
            pl.BlockSpec((_BEM, D), blk),
            pl.BlockSpec((_BEM, D), blk),
            pl.BlockSpec((D, D), cst),
            pl.BlockSpec((1, D), cst),
            pl.BlockSpec((D, D), cst),
            pl.BlockSpec((1, D), cst),
        ],
        out_specs=pl.BlockSpec((_BEM, D), blk),
        out_shape=jax.ShapeDtypeStruct((EP, D), _f32),
    )(gs, ea, w1e, b1, w2, b2)


_CHR = 1000
_NBR = N // _CHR

_DN = DH ** -0.25
_RATIO = M ** -0.5

_EYE8 = np.eye(8, dtype=np.float32)
_R8 = np.kron(_EYE8, np.ones((1, 64), np.float32))
_HD = np.kron(_EYE8, np.ones((16, 1), np.float32))
_S8 = np.kron(_EYE8, np.ones((1, 16), np.float32))
_R8T = np.kron(_EYE8, np.ones((64, 1), np.float32))
_CTXM = (_EYE8[:, None, :, None]
         * np.ones((1, 64, 1, 16), np.float32)).reshape(512, 128)


def _ln(x, g, b):
    mu = jnp.mean(x, axis=-1, keepdims=True)
    va = jnp.mean((x - mu) ** 2, axis=-1, keepdims=True)
    return (x - mu) / jnp.sqrt(va + 1e-5) * g + b


def _mm(a, b):
    return jnp.dot(a, b, preferred_element_type=_f32)


def _phi_raw(hc, t_w, bias, p_mat, hd_mat, r8_mat):
    t = (_mm(hc, t_w) + bias) * _DN
    u = _mm(t, p_mat)
    d2 = _mm(t * t, hd_mat) * 0.5
    return u, _mm(d2, r8_mat)


_CBLK = lambda i: (i, 0)
_CCST = lambda i: (0, 0)


def _kmax_body(h_ref, kt, bk, p_mat, out_ref):
    i = pl.program_id(0)
    t = (_mm(h_ref[...], kt[...]) + bk[...]) * _DN
    m = jnp.max(_mm(t, p_mat[...]))

    @pl.when(i == 0)
    def _():
        out_ref[0, 0] = m

    @pl.when(i > 0)
    def _():
        out_ref[0, 0] = jnp.maximum(out_ref[0, 0], m)


def _kmax_call(h, kt, bk, p_mat):
    return pl.pallas_call(
        _kmax_body,
        grid=(_NBR,),
        in_specs=[
            pl.BlockSpec((_CHR, D), _CBLK),
            pl.BlockSpec((D, D), _CCST),
            pl.BlockSpec((1, D), _CCST),
            pl.BlockSpec((D, 512), _CCST),
        ],
        out_specs=pl.BlockSpec((1, 1), _CCST, memory_space=pltpu.SMEM),
        out_shape=jax.ShapeDtypeStruct((1, 1), _f32),
    )(h, kt, bk, p_mat)


def _ctx_body(h_ref, kt, bk, vt, bv, p_mat, hd_mat, r8_mat, mx_ref,
              ctx_ref, ksum_ref):
    i = pl.program_id(0)
    hc = h_ref[...]
    u, d2b = _phi_raw(hc, kt[...], bk[...], p_mat[...], hd_mat[...],
                      r8_mat[...])
    kp = _RATIO * (jnp.exp(u - d2b - mx_ref[0, 0]) + 1e-4)
    vc = _mm(hc, vt[...]) + bv[...]
    c = lax.dot_general(kp, vc, (((0,), (0,)), ((), ())),
                        preferred_element_type=_f32)
    ks = jnp.sum(kp, axis=0, keepdims=True)

    @pl.when(i == 0)
    def _():
        ctx_ref[...] = c
        ksum_ref[...] = ks

    @pl.when(i > 0)
    def _():
        ctx_ref[...] += c
        ksum_ref[...] += ks


def _ctx_call(h, kt, bk, vt, bv, p_mat, mx):
    return pl.pallas_call(
        _ctx_body,
        grid=(_NBR,),
        in_specs=[
            pl.BlockSpec((_CHR, D), _CBLK),
            pl.BlockSpec((D, D), _CCST),
            pl.BlockSpec((1, D), _CCST),
            pl.BlockSpec((D, D), _CCST),
            pl.BlockSpec((1, D), _CCST),
            pl.BlockSpec((D, 512), _CCST),
            pl.BlockSpec((D, 8), _CCST),
            pl.BlockSpec((8, 512), _CCST),
            pl.BlockSpec((1, 1), _CCST, memory_space=pltpu.SMEM),
        ],
        out_specs=(
            pl.BlockSpec((512, D), _CCST),
            pl.BlockSpec((1, 512), _CCST),
        ),
        out_shape=(
            jax.ShapeDtypeStruct((512, D), _f32),
            jax.ShapeDtypeStruct((1, 512), _f32),
        ),
    )(h, kt, bk, vt, bv, p_mat, jnp.asarray(_HD), jnp.asarray(_R8), mx)


def _main_core(hc, p0c, p1c, ctx, ksum, w):
    (n1a, n1b, bn1, n2t, bn2, ggcn, bgcn, glo, blo,
     qt, bq, ot, bo, gat, bat,
     f1t, bf1, f2t, bf2, gff, bff,
     p_mat, hd_mat, r8_mat, r8t_mat, s8_mat, ctxm_mat) = w
    u, d2b = _phi_raw(hc, qt, bq, p_mat, hd_mat, r8_mat)
    mxq = jnp.concatenate(
        [jnp.max(u[:, hh * 64:(hh + 1) * 64], axis=1, keepdims=True)
         for hh in range(HEADS)], axis=1)
    qp = _RATIO * (jnp.exp(u - d2b - _mm(mxq, r8_mat)) + 1e-4)
    num = _mm(qp, ctx * ctxm_mat)
    den = _mm(qp * ksum, r8t_mat) + 1e-6
    o = num / _mm(den, s8_mat)
    attn = _mm(o, ot) + bo
    h_attn = _ln(hc + attn, gat, bat)

    agg = p0c + p1c
    t = jnp.maximum(_mm(hc, n1a) + _mm(agg, n1b) + bn1, 0.0)
    out = _mm(t, n2t) + bn2 + hc
    h_local = _ln(out, ggcn, bgcn)
    h_local = _ln(hc + h_local, glo, blo)

    hs = h_local + h_attn
    ff = _mm(jnp.maximum(_mm(hs, f1t) + bf1, 0.0), f2t) + bf2
    return _ln(hs + ff, gff, bff)


_NW_DENSE = 27


def _main_mid_body(*refs):
    (h_ref, p0_ref, p1_ref, ctx_ref, ksum_ref) = refs[:5]
    w = tuple(r[...] for r in refs[5:5 + _NW_DENSE])
    w1in, w1jn = refs[5 + _NW_DENSE], refs[6 + _NW_DENSE]
    h_out, a_out, b_out = refs[-3:]
    hn = _main_core(h_ref[...], p0_ref[...], p1_ref[...], ctx_ref[...],
                    ksum_ref[...], w)
    h_out[...] = hn
    a_out[...] = _mm(hn, w1in[...])
    b_out[...] = _mm(hn, w1jn[...])


def _main_last_body(*refs):
    (h_ref, p0_ref, p1_ref, ctx_ref, ksum_ref) = refs[:5]
    w = tuple(r[...] for r in refs[5:5 + _NW_DENSE])
    h1t, bh1, h2t, bh2 = refs[5 + _NW_DENSE:9 + _NW_DENSE]
    y_out = refs[-1]
    hn = _main_core(h_ref[...], p0_ref[...], p1_ref[...], ctx_ref[...],
                    ksum_ref[...], w)
    z = jnp.maximum(_mm(hn, h1t[...]) + bh1[...], 0.0)
    y_out[...] = _mm(z, h2t[...]) + bh2[...]


def _dense_w_specs():
    return [
        pl.BlockSpec((D, D), _CCST),
        pl.BlockSpec((D, D), _CCST),
        pl.BlockSpec((1, D), _CCST),
        pl.BlockSpec((D, D), _CCST),
        pl.BlockSpec((1, D), _CCST),
        pl.BlockSpec((1, D), _CCST),
        pl.BlockSpec((1, D), _CCST),
        pl.BlockSpec((1, D), _CCST),
        pl.BlockSpec((1, D), _CCST),
        pl.BlockSpec((D, D), _CCST),
        pl.BlockSpec((1, D), _CCST),
        pl.BlockSpec((D, D), _CCST),
        pl.BlockSpec((1, D), _CCST),
        pl.BlockSpec((1, D), _CCST),
        pl.BlockSpec((1, D), _CCST),
        pl.BlockSpec((D, 256), _CCST),
        pl.BlockSpec((1, 256), _CCST),
        pl.BlockSpec((256, D), _CCST),
        pl.BlockSpec((1, D), _CCST),
        pl.BlockSpec((1, D), _CCST),
        pl.BlockSpec((1, D), _CCST),
        pl.BlockSpec((D, 512), _CCST),
        pl.BlockSpec((D, 8), _CCST),
        pl.BlockSpec((8, 512), _CCST),
        pl.BlockSpec((512, 8), _CCST),
        pl.BlockSpec((8, D), _CCST),
        pl.BlockSpec((512, D), _CCST),
    ]


def _dense_mid(h, p0, p1, ctx, ksum, wl):
    in_specs = [
        pl.BlockSpec((_CHR, D), _CBLK),
        pl.BlockSpec((_CHR, D), _CBLK),
        pl.BlockSpec((_CHR, D), _CBLK),
        pl.BlockSpec((512, D), _CCST),
        pl.BlockSpec((1, 512), _CCST),
    ] + _dense_w_specs() + [
        pl.BlockSpec((D, D), _CCST),
        pl.BlockSpec((D, D), _CCST),
    ]
    return pl.pallas_call(
        _main_mid_body,
        grid=(_NBR,),
        in_specs=in_specs,
        out_specs=(
            pl.BlockSpec((_CHR, D), _CBLK),
            pl.BlockSpec((_CHR, D), _CBLK),
            pl.BlockSpec((_CHR, D), _CBLK),
        ),
        out_shape=(
            jax.ShapeDtypeStruct((N, D), _f32),
            jax.ShapeDtypeStruct((N, D), _f32),
            jax.ShapeDtypeStruct((N, D), _f32),
        ),
    )(h, p0, p1, ctx, ksum, *wl)


def _dense_last(h, p0, p1, ctx, ksum, wl):
    in_specs = [
        pl.BlockSpec((_CHR, D), _CBLK),
        pl.BlockSpec((_CHR, D), _CBLK),
        pl.BlockSpec((_CHR, D), _CBLK),
        pl.BlockSpec((512, D), _CCST),
        pl.BlockSpec((1, 512), _CCST),
    ] + _dense_w_specs() + [
        pl.BlockSpec((D, D), _CCST),
        pl.BlockSpec((1, D), _CCST),
        pl.BlockSpec((D, 1), _CCST),
        pl.BlockSpec((1, 1), _CCST),
    ]
    return pl.pallas_call(
        _main_last_body,
        grid=(_NBR,),
        in_specs=in_specs,
        out_specs=pl.BlockSpec((_CHR, 1), _CBLK),
        out_shape=jax.ShapeDtypeStruct((N, 1), _f32),
    )(h, p0, p1, ctx, ksum, *wl)


def _row(v):
    return v.reshape(1, -1)


def kernel(x, edge_index, edge_attr, pe, rwse, batch, params):
    p = params
    src = edge_index[0].astype(_i32)
    dst = edge_index[1].astype(_i32)
    pad = EP - E
    zpad = jnp.zeros((pad,), _i32)
    gdst = jnp.concatenate([dst, zpad])
    gsrc = jnp.concatenate([src, zpad])
    sdst = jnp.concatenate([dst, jnp.full((pad,), N, _i32)]).reshape(
        NROW_CH, 1, CHS)
    ea_pad = jnp.concatenate(
        [edge_attr, jnp.zeros((pad, 1), _f32)], axis=0)
    zeros_z = jnp.zeros((ZR, D), _f32)

    proj = p["proj"]
    p_mat = (_EYE8[:, None, :, None]
             * proj.T[None, :, None, :]).reshape(D, 512)

    stats = _stats_call(ea_pad)
    ea = _ea_call(ea_pad, stats,
                  _row(p["enc_edge"]["W"][:, 0]),
                  _row(p["enc_edge_bn"]["g"]), _row(p["enc_edge_bn"]["b"]))

    l0 = p["layers"][0]
    h, a_t, b_t = _enc_call(
        x, rwse,
        _row(p["enc_node"]["W"][:, 0]),
        _row(p["enc_node_bn"]["g"]), _row(p["enc_node_bn"]["b"]),
        p["enc_rwse"]["W"].T, _row(p["enc_rwse"]["b"]),
        _row(p["enc_rwse_bn"]["g"]), _row(p["enc_rwse_bn"]["b"]),
        p["pre"]["W"].T, _row(p["pre"]["b"]),
        _row(p["pre_bn"]["g"]), _row(p["pre_bn"]["b"]),
        l0["e1"]["W"][:, :D].T, l0["e1"]["W"][:, D:2 * D].T,
    )

    out = None
    for li in range(6):
        lp = p["layers"][li]
        gs = _sc_gather(a_t, b_t, gdst, gsrc)
        m = _edge_call(gs, ea,
                       lp["e1"]["W"][:, 2 * D:].T, _row(lp["e1"]["b"]),
                       lp["e2"]["W"].T, _row(lp["e2"]["b"]))
        parts = _sc_scatter(m, sdst, zeros_z)
        p0 = parts[:NP]
        p1 = parts[NP:]

        kt = lp["k"]["W"].T
        bk = _row(lp["k"]["b"])
        mx = _kmax_call(h, kt, bk, p_mat)
        ctx, ksum = _ctx_call(h, kt, bk, lp["v"]["W"].T, _row(lp["v"]["b"]),
                              p_mat, mx)

        wl = [
            lp["n1"]["W"][:, :D].T, lp["n1"]["W"][:, D:].T,
            _row(lp["n1"]["b"]),
            lp["n2"]["W"].T, _row(lp["n2"]["b"]),
            _row(lp["gcn_ln"]["g"]), _row(lp["gcn_ln"]["b"]),
            _row(lp["ln_local"]["g"]), _row(lp["ln_local"]["b"]),
            lp["q"]["W"].T, _row(lp["q"]["b"]),
            lp["o"]["W"].T, _row(lp["o"]["b"]),
            _row(lp["ln_attn"]["g"]), _row(lp["ln_attn"]["b"]),
            lp["f1"]["W"].T, _row(lp["f1"]["b"]),
            lp["f2"]["W"].T, _row(lp["f2"]["b"]),
            _row(lp["ln_ffn"]["g"]), _row(lp["ln_ffn"]["b"]),
            p_mat, _HD, _R8, _R8T, _S8, _CTXM,
        ]
        if li < 5:
            ln = p["layers"][li + 1]
            wl += [ln["e1"]["W"][:, :D].T, ln["e1"]["W"][:, D:2 * D].T]
            h, a_t, b_t = _dense_mid(h, p0, p1, ctx, ksum, wl)
        else:
            wl += [p["head1"]["W"].T, _row(p["head1"]["b"]),
                   p["head2"]["W"].T.reshape(D, 1),
                   _row(p["head2"]["b"]).reshape(1, 1)]
            out = _dense_last(h, p0, p1, ctx, ksum, wl)
    return out
